# baseline (speedup 1.0000x reference)
import functools

import jax
import jax.numpy as jnp
from jax import lax
from jax.experimental import pallas as pl
from jax.experimental.pallas import tpu as pltpu
from jax.experimental.pallas import tpu_sc as plsc

_SEQ = 4096
_HID = 1024
_BLK = 512

_NC = 2
_NS = 16
_ROWS_PER_WORKER = _SEQ // (_NC * _NS)

_sc_mesh = plsc.VectorSubcoreMesh(core_axis_name="c", subcore_axis_name="s")


@functools.partial(
    pl.kernel,
    mesh=_sc_mesh,
    out_type=jax.ShapeDtypeStruct((_SEQ, _HID), jnp.float32),
)
def _sc_copy(src_hbm, out_hbm):
    wid = lax.axis_index("s") * _NC + lax.axis_index("c")
    base = wid * _ROWS_PER_WORKER
    pltpu.sync_copy(
        src_hbm.at[pl.ds(base, _ROWS_PER_WORKER), :],
        out_hbm.at[pl.ds(base, _ROWS_PER_WORKER), :],
    )


def _zero_kernel(attn_ref):
    attn_ref[...] = jnp.zeros_like(attn_ref)


def kernel(hidden_states):
    batch, seq, hid = hidden_states.shape
    h2 = hidden_states.reshape(seq, hid)
    out = _sc_copy(h2)
    attn = pl.pallas_call(
        _zero_kernel,
        grid=(seq // _BLK,),
        in_specs=[],
        out_specs=[pl.BlockSpec((_BLK, seq), lambda i: (i, 0))],
        out_shape=[jax.ShapeDtypeStruct((seq, seq), hidden_states.dtype)],
    )()[0]
    return (out.reshape(batch, seq, hid), attn.reshape(batch, seq, seq))


# speedup vs baseline: 11.0120x; 11.0120x over previous
import functools

import jax
import jax.numpy as jnp
from jax import lax
from jax.experimental import pallas as pl
from jax.experimental.pallas import tpu as pltpu
from jax.experimental.pallas import tpu_sc as plsc

_SEQ = 4096
_HID = 1024
_BLK = 512

_NC = 2
_NS = 16
_ROWS_PER_WORKER = _SEQ // (_NC * _NS)

_sc_mesh = plsc.VectorSubcoreMesh(core_axis_name="c", subcore_axis_name="s")


_CHUNK = 64


@functools.partial(
    pl.kernel,
    mesh=_sc_mesh,
    out_type=jax.ShapeDtypeStruct((_SEQ, _HID), jnp.float32),
    scratch_types=[
        pltpu.VMEM((_CHUNK, _HID), jnp.float32),
        pltpu.SemaphoreType.DMA,
    ],
)
def _sc_copy(src_hbm, out_hbm, buf, sem):
    wid = lax.axis_index("s") * _NC + lax.axis_index("c")
    base = wid * _ROWS_PER_WORKER
    for c in range(_ROWS_PER_WORKER // _CHUNK):
        off = base + c * _CHUNK
        pltpu.async_copy(src_hbm.at[pl.ds(off, _CHUNK), :], buf, sem).wait()
        pltpu.async_copy(buf, out_hbm.at[pl.ds(off, _CHUNK), :], sem).wait()


def _zero_kernel(attn_ref):
    attn_ref[...] = jnp.zeros_like(attn_ref)


def kernel(hidden_states):
    batch, seq, hid = hidden_states.shape
    h2 = hidden_states.reshape(seq, hid)
    out = _sc_copy(h2)
    attn = pl.pallas_call(
        _zero_kernel,
        grid=(seq // _BLK,),
        in_specs=[],
        out_specs=[pl.BlockSpec((_BLK, seq), lambda i: (i, 0))],
        out_shape=[jax.ShapeDtypeStruct((seq, seq), hidden_states.dtype)],
    )()[0]
    return (out.reshape(batch, seq, hid), attn.reshape(batch, seq, seq))


# speedup vs baseline: 11.0438x; 1.0029x over previous
import functools

import jax
import jax.numpy as jnp
from jax import lax
from jax.experimental import pallas as pl
from jax.experimental.pallas import tpu as pltpu
from jax.experimental.pallas import tpu_sc as plsc

_SEQ = 4096
_HID = 1024
_BLK = 512

_NC = 2
_NS = 16
_ROWS_PER_WORKER = _SEQ // (_NC * _NS)

_sc_mesh = plsc.VectorSubcoreMesh(core_axis_name="c", subcore_axis_name="s")


_CHUNK = 32
_NBUF = 3
_NCHUNK = _ROWS_PER_WORKER // _CHUNK


@functools.partial(
    pl.kernel,
    mesh=_sc_mesh,
    out_type=jax.ShapeDtypeStruct((_SEQ, _HID), jnp.float32),
    scratch_types=(
        [pltpu.VMEM((_CHUNK, _HID), jnp.float32) for _ in range(_NBUF)]
        + [pltpu.SemaphoreType.DMA for _ in range(2 * _NBUF)]
    ),
)
def _sc_copy(src_hbm, out_hbm, *scratch):
    bufs = scratch[:_NBUF]
    in_sems = scratch[_NBUF : 2 * _NBUF]
    out_sems = scratch[2 * _NBUF :]
    wid = lax.axis_index("s") * _NC + lax.axis_index("c")
    base = wid * _ROWS_PER_WORKER

    def start_in(c):
        off = base + c * _CHUNK
        pltpu.async_copy(
            src_hbm.at[pl.ds(off, _CHUNK), :], bufs[c % _NBUF], in_sems[c % _NBUF]
        )

    for c in range(min(_NBUF, _NCHUNK)):
        start_in(c)
    for c in range(_NCHUNK):
        b = c % _NBUF
        off = base + c * _CHUNK
        pltpu.make_async_copy(
            src_hbm.at[pl.ds(off, _CHUNK), :], bufs[b], in_sems[b]
        ).wait()
        pltpu.async_copy(bufs[b], out_hbm.at[pl.ds(off, _CHUNK), :], out_sems[b])
        nxt = c + _NBUF
        if nxt < _NCHUNK:
            pltpu.make_async_copy(
                bufs[b], out_hbm.at[pl.ds(off, _CHUNK), :], out_sems[b]
            ).wait()
            start_in(nxt)
    for c in range(max(_NCHUNK - _NBUF, 0), _NCHUNK):
        b = c % _NBUF
        off = base + c * _CHUNK
        pltpu.make_async_copy(
            bufs[b], out_hbm.at[pl.ds(off, _CHUNK), :], out_sems[b]
        ).wait()


def _zero_kernel(attn_ref):
    attn_ref[...] = jnp.zeros_like(attn_ref)


def kernel(hidden_states):
    batch, seq, hid = hidden_states.shape
    h2 = hidden_states.reshape(seq, hid)
    out = _sc_copy(h2)
    attn = pl.pallas_call(
        _zero_kernel,
        grid=(seq // _BLK,),
        in_specs=[],
        out_specs=[pl.BlockSpec((_BLK, seq), lambda i: (i, 0))],
        out_shape=[jax.ShapeDtypeStruct((seq, seq), hidden_states.dtype)],
    )()[0]
    return (out.reshape(batch, seq, hid), attn.reshape(batch, seq, seq))
